# baseline (speedup 1.0000x reference)
import jax
import jax.numpy as jnp
from jax import lax
from jax.experimental import pallas as pl
from jax.experimental.pallas import tpu as pltpu
from jax.experimental.pallas import tpu_sc as plsc

_N_NODES = 10000
_N_EDGES = 320000
_F = 128

_NC = 2
_NS = 16
_NW = _NC * _NS
_EDGES_PER_W = _N_EDGES // _NW
_CHUNK = 128
_NCH = _EDGES_PER_W // _CHUNK
_TAIL = _EDGES_PER_W - _NCH * _CHUNK
_NBUF = 3
_NOUT = _NCH // _NBUF
_N_PAD = 10112
_ROWS_PER_TILE = _N_PAD // _NS
_DRAIN_FULL = _ROWS_PER_TILE // _CHUNK
_DRAIN_REM = _ROWS_PER_TILE - _DRAIN_FULL * _CHUNK


def _seg_sum_body(feat_hbm, dst_hbm, zeros_hbm, out0_hbm, out1_hbm,
                  idx0, idx1, idx2, rows0, rows1, rows2, tidx, acc_sh,
                  sem0, sem1, sem2):
    c = lax.axis_index("c")
    s = lax.axis_index("s")
    wid = c * _NS + s
    e_base = wid * _EDGES_PER_W
    idx = [idx0, idx1, idx2]
    rows = [rows0, rows1, rows2]
    sems = [sem0, sem1, sem2]

    pltpu.sync_copy(zeros_hbm, rows0)
    r_base = s * _ROWS_PER_TILE
    for j in range(_DRAIN_FULL):
        pltpu.sync_copy(rows0, acc_sh.at[pl.ds(r_base + j * _CHUNK, _CHUNK)])
    pltpu.sync_copy(
        rows0.at[pl.ds(0, _DRAIN_REM)],
        acc_sh.at[pl.ds(r_base + _DRAIN_FULL * _CHUNK, _DRAIN_REM)])
    plsc.subcore_barrier()

    def start(b, g):
        off = e_base + g * _CHUNK
        pltpu.async_copy(dst_hbm.at[pl.ds(_N_EDGES + off, _CHUNK)], idx[b], sems[b])
        pltpu.async_copy(feat_hbm.at[pl.ds(off, _CHUNK), :], rows[b], sems[b])

    def wait(b, g):
        off = e_base + g * _CHUNK
        pltpu.make_async_copy(dst_hbm.at[pl.ds(_N_EDGES + off, _CHUNK)], idx[b], sems[b]).wait()
        pltpu.make_async_copy(feat_hbm.at[pl.ds(off, _CHUNK), :], rows[b], sems[b]).wait()

    for b in range(_NBUF):
        start(b, b)

    def outer(og, carry):
        for b in range(_NBUF):
            g = og * _NBUF + b
            wait(b, g)
            pltpu.sync_copy(rows[b], acc_sh.at[idx[b]], add=True)

            @pl.when(og < _NOUT - 1)
            def _():
                start(b, g + _NBUF)

        return carry

    lax.fori_loop(0, _NOUT, outer, 0)

    t_off = e_base + _NCH * _CHUNK
    pltpu.sync_copy(dst_hbm.at[pl.ds(_N_EDGES + t_off, _TAIL)], tidx)
    pltpu.sync_copy(feat_hbm.at[pl.ds(t_off, _TAIL), :], rows0.at[pl.ds(0, _TAIL)])
    pltpu.sync_copy(rows0.at[pl.ds(0, _TAIL)], acc_sh.at[tidx], add=True)

    plsc.subcore_barrier()

    def drain(out_hbm):
        for j in range(_DRAIN_FULL):
            b = j % 2
            off = r_base + j * _CHUNK
            if j >= 2:
                poff = r_base + (j - 2) * _CHUNK
                pltpu.make_async_copy(rows[b], out_hbm.at[pl.ds(poff, _CHUNK), :], sems[b]).wait()
            pltpu.sync_copy(acc_sh.at[pl.ds(off, _CHUNK)], rows[b])
            pltpu.async_copy(rows[b], out_hbm.at[pl.ds(off, _CHUNK), :], sems[b])
        for j in range(_DRAIN_FULL - 2, _DRAIN_FULL):
            b = j % 2
            off = r_base + j * _CHUNK
            pltpu.make_async_copy(rows[b], out_hbm.at[pl.ds(off, _CHUNK), :], sems[b]).wait()
        off = r_base + _DRAIN_FULL * _CHUNK
        pltpu.sync_copy(acc_sh.at[pl.ds(off, _DRAIN_REM)], rows0.at[pl.ds(0, _DRAIN_REM)])
        pltpu.sync_copy(rows0.at[pl.ds(0, _DRAIN_REM)], out_hbm.at[pl.ds(off, _DRAIN_REM), :])

    @pl.when(c == 0)
    def _():
        drain(out0_hbm)

    @pl.when(c == 1)
    def _():
        drain(out1_hbm)


@jax.jit
def _segment_sum_sc(feat, dst, zeros):
    mesh = plsc.VectorSubcoreMesh(core_axis_name="c", subcore_axis_name="s")
    run = pl.kernel(
        _seg_sum_body,
        out_type=(jax.ShapeDtypeStruct((_N_PAD, _F), jnp.float32),
                  jax.ShapeDtypeStruct((_N_PAD, _F), jnp.float32)),
        mesh=mesh,
        compiler_params=pltpu.CompilerParams(use_tc_tiling_on_sc=False),
        scratch_types=[
            pltpu.VMEM((_CHUNK,), jnp.int32),
            pltpu.VMEM((_CHUNK,), jnp.int32),
            pltpu.VMEM((_CHUNK,), jnp.int32),
            pltpu.VMEM((_CHUNK, _F), jnp.float32),
            pltpu.VMEM((_CHUNK, _F), jnp.float32),
            pltpu.VMEM((_CHUNK, _F), jnp.float32),
            pltpu.VMEM((_TAIL,), jnp.int32),
            pltpu.VMEM_SHARED((_N_PAD, _F), jnp.float32),
            pltpu.SemaphoreType.DMA,
            pltpu.SemaphoreType.DMA,
            pltpu.SemaphoreType.DMA,
        ],
    )
    return run(feat, dst, zeros)


def _linear_relu_body(p0_ref, p1_ref, wt_ref, b_ref, o_ref):
    x = p0_ref[...] + p1_ref[...]
    y = jnp.dot(x, wt_ref[...], preferred_element_type=jnp.float32)
    o_ref[...] = jnp.maximum(y + b_ref[...], 0.0)


_ROWS_BLK = 2000
_N_BLKS = _N_NODES // _ROWS_BLK


@jax.jit
def _linear_relu_tc(p0, p1, wt, b2d):
    return pl.pallas_call(
        _linear_relu_body,
        grid=(_N_BLKS,),
        in_specs=[
            pl.BlockSpec((_ROWS_BLK, _F), lambda i: (i, 0)),
            pl.BlockSpec((_ROWS_BLK, _F), lambda i: (i, 0)),
            pl.BlockSpec((_F, _F), lambda i: (0, 0)),
            pl.BlockSpec((1, _F), lambda i: (0, 0)),
        ],
        out_specs=pl.BlockSpec((_ROWS_BLK, _F), lambda i: (i, 0)),
        out_shape=jax.ShapeDtypeStruct((_N_NODES, _F), jnp.float32),
    )(p0, p1, wt, b2d)


def kernel(feat, edge_index, W, b):
    ei_flat = edge_index.astype(jnp.int32).reshape(-1)
    zeros = jnp.zeros((_CHUNK, _F), jnp.float32)
    p0, p1 = _segment_sum_sc(feat, ei_flat, zeros)
    return _linear_relu_tc(p0, p1, W.T, b.reshape(1, _F))


# speedup vs baseline: 1.0002x; 1.0002x over previous
import jax
import jax.numpy as jnp
from jax import lax
from jax.experimental import pallas as pl
from jax.experimental.pallas import tpu as pltpu
from jax.experimental.pallas import tpu_sc as plsc

_N_NODES = 10000
_N_EDGES = 320000
_F = 128

_NC = 2
_NS = 16
_NW = _NC * _NS
_EDGES_PER_W = _N_EDGES // _NW
_CHUNK = 128
_NCH = _EDGES_PER_W // _CHUNK
_TAIL = _EDGES_PER_W - _NCH * _CHUNK
_NBUF = 3
_NOUT = _NCH // _NBUF
_N_PAD = 10112
_ROWS_PER_TILE = _N_PAD // _NS
_DRAIN_FULL = _ROWS_PER_TILE // _CHUNK
_DRAIN_REM = _ROWS_PER_TILE - _DRAIN_FULL * _CHUNK


def _seg_sum_body(feat_hbm, dst_hbm, zeros_hbm, out0_hbm, out1_hbm,
                  idx0, idx1, idx2, rows0, rows1, rows2, tidx, acc_sh,
                  sem0, sem1, sem2):
    c = lax.axis_index("c")
    s = lax.axis_index("s")
    wid = c * _NS + s
    e_base = wid * _EDGES_PER_W
    idx = [idx0, idx1, idx2]
    rows = [rows0, rows1, rows2]
    sems = [sem0, sem1, sem2]

    pltpu.sync_copy(zeros_hbm, rows0)
    r_base = s * _ROWS_PER_TILE
    for j in range(_DRAIN_FULL):
        pltpu.sync_copy(rows0, acc_sh.at[pl.ds(r_base + j * _CHUNK, _CHUNK)])
    pltpu.sync_copy(
        rows0.at[pl.ds(0, _DRAIN_REM)],
        acc_sh.at[pl.ds(r_base + _DRAIN_FULL * _CHUNK, _DRAIN_REM)])
    plsc.subcore_barrier()

    def start(b, g):
        off = e_base + g * _CHUNK
        pltpu.async_copy(dst_hbm.at[1, pl.ds(off, _CHUNK)], idx[b], sems[b])
        pltpu.async_copy(feat_hbm.at[pl.ds(off, _CHUNK), :], rows[b], sems[b])

    def wait(b, g):
        off = e_base + g * _CHUNK
        pltpu.make_async_copy(dst_hbm.at[1, pl.ds(off, _CHUNK)], idx[b], sems[b]).wait()
        pltpu.make_async_copy(feat_hbm.at[pl.ds(off, _CHUNK), :], rows[b], sems[b]).wait()

    for b in range(_NBUF):
        start(b, b)

    def outer(og, carry):
        for b in range(_NBUF):
            g = og * _NBUF + b
            wait(b, g)
            pltpu.sync_copy(rows[b], acc_sh.at[idx[b]], add=True)

            @pl.when(og < _NOUT - 1)
            def _():
                start(b, g + _NBUF)

        return carry

    lax.fori_loop(0, _NOUT, outer, 0)

    t_off = e_base + _NCH * _CHUNK
    pltpu.sync_copy(dst_hbm.at[1, pl.ds(t_off, _TAIL)], tidx)
    pltpu.sync_copy(feat_hbm.at[pl.ds(t_off, _TAIL), :], rows0.at[pl.ds(0, _TAIL)])
    pltpu.sync_copy(rows0.at[pl.ds(0, _TAIL)], acc_sh.at[tidx], add=True)

    plsc.subcore_barrier()

    def drain(out_hbm):
        for j in range(_DRAIN_FULL):
            b = j % 2
            off = r_base + j * _CHUNK
            if j >= 2:
                poff = r_base + (j - 2) * _CHUNK
                pltpu.make_async_copy(rows[b], out_hbm.at[pl.ds(poff, _CHUNK), :], sems[b]).wait()
            pltpu.sync_copy(acc_sh.at[pl.ds(off, _CHUNK)], rows[b])
            pltpu.async_copy(rows[b], out_hbm.at[pl.ds(off, _CHUNK), :], sems[b])
        for j in range(_DRAIN_FULL - 2, _DRAIN_FULL):
            b = j % 2
            off = r_base + j * _CHUNK
            pltpu.make_async_copy(rows[b], out_hbm.at[pl.ds(off, _CHUNK), :], sems[b]).wait()
        off = r_base + _DRAIN_FULL * _CHUNK
        pltpu.sync_copy(acc_sh.at[pl.ds(off, _DRAIN_REM)], rows0.at[pl.ds(0, _DRAIN_REM)])
        pltpu.sync_copy(rows0.at[pl.ds(0, _DRAIN_REM)], out_hbm.at[pl.ds(off, _DRAIN_REM), :])

    @pl.when(c == 0)
    def _():
        drain(out0_hbm)

    @pl.when(c == 1)
    def _():
        drain(out1_hbm)


@jax.jit
def _segment_sum_sc(feat, dst, zeros):
    mesh = plsc.VectorSubcoreMesh(core_axis_name="c", subcore_axis_name="s")
    run = pl.kernel(
        _seg_sum_body,
        out_type=(jax.ShapeDtypeStruct((_N_PAD, _F), jnp.float32),
                  jax.ShapeDtypeStruct((_N_PAD, _F), jnp.float32)),
        mesh=mesh,
        compiler_params=pltpu.CompilerParams(use_tc_tiling_on_sc=False),
        scratch_types=[
            pltpu.VMEM((_CHUNK,), jnp.int32),
            pltpu.VMEM((_CHUNK,), jnp.int32),
            pltpu.VMEM((_CHUNK,), jnp.int32),
            pltpu.VMEM((_CHUNK, _F), jnp.float32),
            pltpu.VMEM((_CHUNK, _F), jnp.float32),
            pltpu.VMEM((_CHUNK, _F), jnp.float32),
            pltpu.VMEM((_TAIL,), jnp.int32),
            pltpu.VMEM_SHARED((_N_PAD, _F), jnp.float32),
            pltpu.SemaphoreType.DMA,
            pltpu.SemaphoreType.DMA,
            pltpu.SemaphoreType.DMA,
        ],
    )
    return run(feat, dst, zeros)


def _linear_relu_body(p0_ref, p1_ref, wt_ref, b_ref, o_ref):
    x = p0_ref[...] + p1_ref[...]
    y = jnp.dot(x, wt_ref[...], preferred_element_type=jnp.float32)
    o_ref[...] = jnp.maximum(y + b_ref[...], 0.0)


_ROWS_BLK = 2000
_N_BLKS = _N_NODES // _ROWS_BLK


@jax.jit
def _linear_relu_tc(p0, p1, wt, b2d):
    return pl.pallas_call(
        _linear_relu_body,
        grid=(_N_BLKS,),
        in_specs=[
            pl.BlockSpec((_ROWS_BLK, _F), lambda i: (i, 0)),
            pl.BlockSpec((_ROWS_BLK, _F), lambda i: (i, 0)),
            pl.BlockSpec((_F, _F), lambda i: (0, 0)),
            pl.BlockSpec((1, _F), lambda i: (0, 0)),
        ],
        out_specs=pl.BlockSpec((_ROWS_BLK, _F), lambda i: (i, 0)),
        out_shape=jax.ShapeDtypeStruct((_N_NODES, _F), jnp.float32),
    )(p0, p1, wt, b2d)


def kernel(feat, edge_index, W, b):
    zeros = jnp.zeros((_CHUNK, _F), jnp.float32)
    p0, p1 = _segment_sum_sc(feat, edge_index.astype(jnp.int32), zeros)
    return _linear_relu_tc(p0, p1, W.T, b.reshape(1, _F))


# speedup vs baseline: 1.0004x; 1.0002x over previous
import jax
import jax.numpy as jnp
from jax import lax
from jax.experimental import pallas as pl
from jax.experimental.pallas import tpu as pltpu
from jax.experimental.pallas import tpu_sc as plsc

_N_NODES = 10000
_N_EDGES = 320000
_F = 128

_NC = 2
_NS = 16
_NW = _NC * _NS
_EDGES_PER_W = _N_EDGES // _NW
_CHUNK = 128
_NCH = _EDGES_PER_W // _CHUNK
_TAIL = _EDGES_PER_W - _NCH * _CHUNK
_NBUF = 3
_NOUT = _NCH // _NBUF
_N_PAD = 10112
_ROWS_PER_TILE = _N_PAD // _NS
_DRAIN_FULL = _ROWS_PER_TILE // _CHUNK
_DRAIN_REM = _ROWS_PER_TILE - _DRAIN_FULL * _CHUNK


def _seg_sum_body(feat_hbm, dst_hbm, zeros_hbm, out0_hbm, out1_hbm,
                  idx0, idx1, idx2, rows0, rows1, rows2, tidx, acc_sh,
                  sem0, sem1, sem2):
    c = lax.axis_index("c")
    s = lax.axis_index("s")
    wid = c * _NS + s
    e_base = wid * _EDGES_PER_W
    idx = [idx0, idx1, idx2]
    rows = [rows0, rows1, rows2]
    sems = [sem0, sem1, sem2]

    pltpu.sync_copy(zeros_hbm, rows0)
    r_base = s * _ROWS_PER_TILE
    for j in range(_DRAIN_FULL):
        pltpu.sync_copy(rows0, acc_sh.at[pl.ds(r_base + j * _CHUNK, _CHUNK)])
    pltpu.sync_copy(
        rows0.at[pl.ds(0, _DRAIN_REM)],
        acc_sh.at[pl.ds(r_base + _DRAIN_FULL * _CHUNK, _DRAIN_REM)])
    plsc.subcore_barrier()

    def start(b, g):
        off = e_base + g * _CHUNK
        pltpu.async_copy(dst_hbm.at[1, pl.ds(off, _CHUNK)], idx[b], sems[b])
        pltpu.async_copy(feat_hbm.at[pl.ds(off, _CHUNK), :], rows[b], sems[b])

    def wait(b, g):
        off = e_base + g * _CHUNK
        pltpu.make_async_copy(dst_hbm.at[1, pl.ds(off, _CHUNK)], idx[b], sems[b]).wait()
        pltpu.make_async_copy(feat_hbm.at[pl.ds(off, _CHUNK), :], rows[b], sems[b]).wait()

    for b in range(_NBUF):
        start(b, b)

    def outer(og, carry):
        for b in range(_NBUF):
            g = og * _NBUF + b
            wait(b, g)
            pltpu.sync_copy(rows[b], acc_sh.at[idx[b]], add=True)

            @pl.when(og < _NOUT - 1)
            def _():
                start(b, g + _NBUF)

        return carry

    lax.fori_loop(0, _NOUT, outer, 0)

    t_off = e_base + _NCH * _CHUNK
    pltpu.sync_copy(dst_hbm.at[1, pl.ds(t_off, _TAIL)], tidx)
    pltpu.sync_copy(feat_hbm.at[pl.ds(t_off, _TAIL), :], rows0.at[pl.ds(0, _TAIL)])
    pltpu.sync_copy(rows0.at[pl.ds(0, _TAIL)], acc_sh.at[tidx], add=True)

    plsc.subcore_barrier()

    def drain(out_hbm):
        for j in range(_DRAIN_FULL):
            b = j % 2
            off = r_base + j * _CHUNK
            if j >= 2:
                poff = r_base + (j - 2) * _CHUNK
                pltpu.make_async_copy(rows[b], out_hbm.at[pl.ds(poff, _CHUNK), :], sems[b]).wait()
            pltpu.sync_copy(acc_sh.at[pl.ds(off, _CHUNK)], rows[b])
            pltpu.async_copy(rows[b], out_hbm.at[pl.ds(off, _CHUNK), :], sems[b])
        for j in range(_DRAIN_FULL - 2, _DRAIN_FULL):
            b = j % 2
            off = r_base + j * _CHUNK
            pltpu.make_async_copy(rows[b], out_hbm.at[pl.ds(off, _CHUNK), :], sems[b]).wait()
        off = r_base + _DRAIN_FULL * _CHUNK
        pltpu.sync_copy(acc_sh.at[pl.ds(off, _DRAIN_REM)], rows0.at[pl.ds(0, _DRAIN_REM)])
        pltpu.sync_copy(rows0.at[pl.ds(0, _DRAIN_REM)], out_hbm.at[pl.ds(off, _DRAIN_REM), :])

    @pl.when(c == 0)
    def _():
        drain(out0_hbm)

    @pl.when(c == 1)
    def _():
        drain(out1_hbm)


@jax.jit
def _segment_sum_sc(feat, dst, zeros):
    mesh = plsc.VectorSubcoreMesh(core_axis_name="c", subcore_axis_name="s")
    run = pl.kernel(
        _seg_sum_body,
        out_type=(jax.ShapeDtypeStruct((_N_PAD, _F), jnp.float32),
                  jax.ShapeDtypeStruct((_N_PAD, _F), jnp.float32)),
        mesh=mesh,
        compiler_params=pltpu.CompilerParams(use_tc_tiling_on_sc=False),
        scratch_types=[
            pltpu.VMEM((_CHUNK,), jnp.int32),
            pltpu.VMEM((_CHUNK,), jnp.int32),
            pltpu.VMEM((_CHUNK,), jnp.int32),
            pltpu.VMEM((_CHUNK, _F), jnp.float32),
            pltpu.VMEM((_CHUNK, _F), jnp.float32),
            pltpu.VMEM((_CHUNK, _F), jnp.float32),
            pltpu.VMEM((_TAIL,), jnp.int32),
            pltpu.VMEM_SHARED((_N_PAD, _F), jnp.float32),
            pltpu.SemaphoreType.DMA,
            pltpu.SemaphoreType.DMA,
            pltpu.SemaphoreType.DMA,
        ],
    )
    return run(feat, dst, zeros)


def _linear_relu_body(p0_ref, p1_ref, wt_ref, b_ref, o_ref):
    x = p0_ref[...] + p1_ref[...]
    y = jnp.dot(x, wt_ref[...], preferred_element_type=jnp.float32)
    o_ref[...] = jnp.maximum(y + b_ref[...], 0.0)


_ROWS_BLK = 2000
_N_BLKS = _N_NODES // _ROWS_BLK


@jax.jit
def _linear_relu_tc(p0, p1, wt, b2d):
    return pl.pallas_call(
        _linear_relu_body,
        grid=(_N_BLKS,),
        in_specs=[
            pl.BlockSpec((_ROWS_BLK, _F), lambda i: (i, 0)),
            pl.BlockSpec((_ROWS_BLK, _F), lambda i: (i, 0)),
            pl.BlockSpec((_F, _F), lambda i: (0, 0)),
            pl.BlockSpec((1, _F), lambda i: (0, 0)),
        ],
        out_specs=pl.BlockSpec((_ROWS_BLK, _F), lambda i: (i, 0)),
        out_shape=jax.ShapeDtypeStruct((_N_NODES, _F), jnp.float32),
    )(p0, p1, wt, b2d)


def kernel(feat, edge_index, W, b):
    if edge_index.dtype != jnp.int32:
        edge_index = edge_index.astype(jnp.int32)
    zeros = jnp.zeros((_CHUNK, _F), jnp.float32)
    p0, p1 = _segment_sum_sc(feat, edge_index, zeros)
    return _linear_relu_tc(p0, p1, W.T, b.reshape(1, _F))
